# baseline (speedup 1.0000x reference)
import functools

import numpy as np
import jax
import jax.numpy as jnp
from jax import lax
from jax.experimental import pallas as pl
from jax.experimental.pallas import tpu as pltpu
from jax.experimental.pallas import tpu_sc as plsc

N = 50000
E = 800000
NP = 50176
NC, NS = 2, 16
NW = NC * NS
EC = E // NW
B = NP // 256
CAPW = 27136
SENT = int(np.int32(np.uint32((0xFFFF << 16) | N)))
ROWB = 1024
GRID = NP // ROWB


def _sc_mesh():
    return plsc.VectorSubcoreMesh(
        core_axis_name="c", subcore_axis_name="s",
        num_cores=NC, num_subcores=NS)


def _wid():
    return lax.axis_index("s") * NC + lax.axis_index("c")


def _make_bin():
    G = (EC + 15) // 16

    @functools.partial(
        pl.kernel, mesh=_sc_mesh(),
        out_type=(jax.ShapeDtypeStruct((NW, 256), jnp.int32),
                  jax.ShapeDtypeStruct((NW * CAPW,), jnp.int32)),
        scratch_types=[
            pltpu.VMEM((EC + 8,), jnp.int32),
            pltpu.VMEM((EC + 8,), jnp.int32),
            pltpu.VMEM((CAPW,), jnp.int32),
            pltpu.VMEM((256,), jnp.int32),
            pltpu.SMEM((256,), jnp.int32),
            pltpu.SMEM((256,), jnp.int32),
        ],
    )
    def bink(src_hbm, dst_hbm, cnt_hbm, packed_hbm,
             sstage, dstage, wordbuf, cntv, cnt, cur):
        w = _wid()
        lanes = lax.iota(jnp.int32, 16)
        pltpu.sync_copy(src_hbm.at[pl.ds(w * EC, EC)], sstage.at[pl.ds(0, EC)])
        pltpu.sync_copy(dst_hbm.at[pl.ds(w * EC, EC)], dstage.at[pl.ds(0, EC)])

        def zc(j, _):
            cnt[j] = 0
            return 0
        lax.fori_loop(0, 256, zc, 0)

        def count(g, _):
            d16 = dstage[pl.ds(g * 16, 16)]
            valid = (g * 16 + lanes) < EC
            b16 = jnp.where(valid, lax.shift_right_logical(d16, 8), 196)
            for lane in range(16):
                bb = b16[lane]
                cnt[bb] = cnt[bb] + 1
            return 0
        lax.fori_loop(0, G, count, 0)

        def mkstart(bb, running):
            cur[bb] = running
            return running + ((cnt[bb] + 7) & ~7)
        lax.fori_loop(0, 197, mkstart, jnp.int32(0))

        def emitc(g, _):
            v = jnp.zeros((16,), jnp.int32)
            for lane in range(16):
                v = jnp.where(lanes == lane, cnt[g * 16 + lane], v)
            cntv[pl.ds(g * 16, 16)] = v
            return 0
        lax.fori_loop(0, 16, emitc, 0)
        pltpu.sync_copy(cntv, cnt_hbm.at[w])

        sent = jnp.full((16,), SENT, jnp.int32)

        def fill(j, _):
            wordbuf[pl.ds(j * 16, 16)] = sent
            return 0
        lax.fori_loop(0, CAPW // 16, fill, 0)

        def place(g, _):
            s16 = sstage[pl.ds(g * 16, 16)]
            d16 = dstage[pl.ds(g * 16, 16)]
            valid = (g * 16 + lanes) < EC
            d16 = jnp.where(valid, d16, 0xFFFF)
            s16 = jnp.where(valid, s16, N)
            word = lax.shift_left(d16, 16) | s16
            b16 = jnp.minimum(lax.shift_right_logical(d16, 8), 196)
            for lane in range(16):
                bb = b16[lane]
                cu = cur[bb]
                base = cu & ~15
                vec = wordbuf[pl.ds(base, 16)]
                vec = jnp.where(lanes == (cu & 15), word[lane], vec)
                wordbuf[pl.ds(base, 16)] = vec
                cur[bb] = cu + 1
            return 0
        lax.fori_loop(0, G, place, 0)

        pltpu.sync_copy(wordbuf, packed_hbm.at[pl.ds(w * CAPW, CAPW)])

    return bink


def _emit_starts(cbuf, w, st7, ln7):
    def per_tile(t, _):
        def per_group(g, running):
            vec = cbuf[t, pl.ds(g * 16, 16)]
            rv = (vec + 7) & ~7
            for lane in range(16):
                bb = g * 16 + lane

                @pl.when((bb < B) & ((bb & 31) == w))
                def _():
                    kk = bb // 32
                    st7[t * 8 + kk] = running
                    ln7[t * 8 + kk] = vec[lane]
                running = running + rv[lane]
            return running
        lax.fori_loop(0, 16, per_group, jnp.int32(0))
        return 0
    lax.fori_loop(0, NW, per_tile, 0)


def _make_deg():
    CH = 512

    @functools.partial(
        pl.kernel, mesh=_sc_mesh(),
        out_type=jax.ShapeDtypeStruct((NP,), jnp.float32),
        scratch_types=[
            pltpu.VMEM((NW, 256), jnp.int32),
            pltpu.VMEM((CH,), jnp.int32),
            pltpu.VMEM((256,), jnp.float32),
            pltpu.SMEM((256,), jnp.int32),
            pltpu.SMEM((256,), jnp.int32),
            pltpu.SMEM((264,), jnp.int32),
        ],
    )
    def degk(cnt_hbm, packed_hbm, deg_hbm, cbuf, wbuf, degbuf, st7, ln7, dcnt):
        w = _wid()
        lanes = lax.iota(jnp.int32, 16)
        pltpu.sync_copy(cnt_hbm, cbuf)
        _emit_starts(cbuf, w, st7, ln7)

        def per_k(kk, _):
            b = w + 32 * kk

            @pl.when(b < B)
            def _():
                def zz(j, _2):
                    dcnt[j] = 0
                    return 0
                lax.fori_loop(0, 264, zz, 0)

                def per_t(t, _2):
                    st = st7[t * 8 + kk]
                    ln = ln7[t * 8 + kk]
                    nch = (ln + CH - 1) // CH

                    def per_ch(ch, _3):
                        off = pl.multiple_of(t * CAPW + st + ch * CH, 8)
                        pltpu.sync_copy(packed_hbm.at[pl.ds(off, CH)], wbuf)

                        def per_g(g, _4):
                            wv = wbuf[pl.ds(g * 16, 16)]
                            dv = lax.shift_right_logical(wv, 16)
                            ok = lax.shift_right_logical(dv, 8) == b
                            dl = jnp.where(ok, dv & 255, 256)
                            for lane in range(16):
                                dd = dl[lane]
                                dcnt[dd] = dcnt[dd] + 1
                            return 0
                        lax.fori_loop(0, CH // 16, per_g, 0)
                        return 0
                    lax.fori_loop(0, nch, per_ch, 0)
                    return 0
                lax.fori_loop(0, NW, per_t, 0)

                def emit(g, _2):
                    v = jnp.zeros((16,), jnp.int32)
                    for lane in range(16):
                        v = jnp.where(lanes == lane, dcnt[g * 16 + lane], v)
                    degbuf[pl.ds(g * 16, 16)] = v.astype(jnp.float32)
                    return 0
                lax.fori_loop(0, 16, emit, 0)
                pltpu.sync_copy(degbuf, deg_hbm.at[pl.ds(b * 256, 256)])
            return 0
        lax.fori_loop(0, 7, per_k, 0)

    return degk


def _make_prop(F):
    K = 128

    @functools.partial(
        pl.kernel, mesh=_sc_mesh(),
        out_type=jax.ShapeDtypeStruct((NP, F), jnp.float32),
        scratch_types=[
            pltpu.VMEM((NW, 256), jnp.int32),
            pltpu.VMEM((K,), jnp.int32),
            pltpu.VMEM((K,), jnp.int32),
            pltpu.VMEM((K,), jnp.int32),
            pltpu.VMEM((K, F), jnp.float32),
            pltpu.VMEM((264, F), jnp.float32),
            pltpu.SMEM((256,), jnp.int32),
            pltpu.SMEM((256,), jnp.int32),
            pltpu.SemaphoreType.DMA,
        ],
    )
    def propk(h_hbm, cnt_hbm, packed_hbm, out_hbm,
              cbuf, wbuf, gidx, dlbuf, rowbuf, acc, st7, ln7, sem):
        w = _wid()
        pltpu.sync_copy(cnt_hbm, cbuf)
        _emit_starts(cbuf, w, st7, ln7)
        zf = jnp.zeros((16,), jnp.float32)

        def per_k(kk, _):
            b = w + 32 * kk

            @pl.when(b < B)
            def _():
                def za(j, _2):
                    for t in range(F // 16):
                        acc[j, pl.ds(t * 16, 16)] = zf
                    return 0
                lax.fori_loop(0, 264, za, 0)

                def per_t(t, _2):
                    st = st7[t * 8 + kk]
                    ln = ln7[t * 8 + kk]
                    nch = (ln + K - 1) // K

                    def per_ch(ch, _3):
                        off = pl.multiple_of(t * CAPW + st + ch * K, 8)
                        pltpu.sync_copy(packed_hbm.at[pl.ds(off, K)], wbuf)

                        def unpack(g, _4):
                            wv = wbuf[pl.ds(g * 16, 16)]
                            dv = lax.shift_right_logical(wv, 16)
                            ok = lax.shift_right_logical(dv, 8) == b
                            gidx[pl.ds(g * 16, 16)] = jnp.where(
                                ok, wv & 0xFFFF, N)
                            dlbuf[pl.ds(g * 16, 16)] = jnp.where(
                                ok, dv & 255, 256)
                            return 0
                        lax.fori_loop(0, K // 16, unpack, 0)

                        pltpu.async_copy(h_hbm.at[gidx], rowbuf, sem).wait()

                        def rmw(g, _4):
                            dl = dlbuf[pl.ds(g * 16, 16)]
                            for lane in range(16):
                                r = dl[lane]
                                for t2 in range(F // 16):
                                    sl = pl.ds(t2 * 16, 16)
                                    acc[r, sl] = acc[r, sl] + \
                                        rowbuf[g * 16 + lane, sl]
                            return 0
                        lax.fori_loop(0, K // 16, rmw, 0)
                        return 0
                    lax.fori_loop(0, nch, per_ch, 0)
                    return 0
                lax.fori_loop(0, NW, per_t, 0)

                pltpu.sync_copy(acc.at[pl.ds(0, 256)],
                                out_hbm.at[pl.ds(b * 256, 256)])
            return 0
        lax.fori_loop(0, 7, per_k, 0)

    return propk


def _rowspec(cols):
    return pl.BlockSpec((ROWB, cols), lambda i: (i, 0))


def _full(shape):
    return pl.BlockSpec(shape, lambda i: (0, 0))


def _rowmask(i):
    idx = lax.broadcasted_iota(jnp.int32, (ROWB, 1), 0) + i * ROWB
    return (idx < N).astype(jnp.float32)


def _dis_kernel(deg):
    def body(d_ref, o_ref):
        o_ref[...] = lax.rsqrt(d_ref[...] + 1.0)
    return pl.pallas_call(
        body, grid=(GRID,),
        in_specs=[_rowspec(1)],
        out_specs=_rowspec(1),
        out_shape=jax.ShapeDtypeStruct((NP, 1), jnp.float32))(deg)


def _mm_first(x, W, dis):
    def body(x_ref, w_ref, d_ref, o_ref):
        i = pl.program_id(0)
        h = jnp.dot(x_ref[...], w_ref[...],
                    preferred_element_type=jnp.float32)
        o_ref[...] = h * d_ref[...] * _rowmask(i)
    fi, fo = W.shape
    return pl.pallas_call(
        body, grid=(GRID,),
        in_specs=[_rowspec(fi), _full((fi, fo)), _rowspec(1)],
        out_specs=_rowspec(fo),
        out_shape=jax.ShapeDtypeStruct((NP, fo), jnp.float32))(x, W, dis)


def _mm_mid(accE, h, dis, b, W):
    def body(a_ref, h_ref, d_ref, b_ref, w_ref, o_ref):
        i = pl.program_id(0)
        z = jax.nn.relu((a_ref[...] + h_ref[...]) * d_ref[...] + b_ref[...])
        o_ref[...] = jnp.dot(z, w_ref[...],
                             preferred_element_type=jnp.float32) \
            * d_ref[...] * _rowmask(i)
    fi, fo = W.shape
    return pl.pallas_call(
        body, grid=(GRID,),
        in_specs=[_rowspec(fi), _rowspec(fi), _rowspec(1),
                  _full((1, fi)), _full((fi, fo))],
        out_specs=_rowspec(fo),
        out_shape=jax.ShapeDtypeStruct((NP, fo), jnp.float32))(accE, h, dis, b, W)


def _finish(accE, h, dis, b):
    def body(a_ref, h_ref, d_ref, b_ref, o_ref):
        v = jax.nn.relu(
            (a_ref[...] + h_ref[...]) * d_ref[...] + b_ref[...])
        o_ref[...] = v[:, :64]
    f = h.shape[1]
    return pl.pallas_call(
        body, grid=(GRID,),
        in_specs=[_rowspec(f), _rowspec(f), _rowspec(1), _full((1, f))],
        out_specs=_rowspec(64),
        out_shape=jax.ShapeDtypeStruct((NP, 64), jnp.float32))(accE, h, dis, b)


_bin = _make_bin()
_deg = _make_deg()
_prop256 = _make_prop(256)
_prop128 = _make_prop(128)


def kernel(x, edge_index, W1, b1, W2, b2, W3, b3):
    src = edge_index[0].astype(jnp.int32)
    dst = edge_index[1].astype(jnp.int32)
    xp = jnp.pad(x, ((0, NP - N), (0, 0)))
    W3p = jnp.pad(W3, ((0, 0), (0, 64)))
    b3p = jnp.pad(b3, (0, 64))

    cnts, packed = _bin(src, dst)
    deg = _deg(cnts, packed).reshape(NP, 1)
    dis = _dis_kernel(deg)

    h1 = _mm_first(xp, W1, dis)
    a1 = _prop256(h1, cnts, packed)
    h2 = _mm_mid(a1, h1, dis, b1.reshape(1, -1), W2)
    a2 = _prop128(h2, cnts, packed)
    h3 = _mm_mid(a2, h2, dis, b2.reshape(1, -1), W3p)
    a3 = _prop128(h3, cnts, packed)
    out = _finish(a3, h3, dis, b3p.reshape(1, -1))
    return out[:N]


# speedup vs baseline: 4.0792x; 4.0792x over previous
import functools

import numpy as np
import jax
import jax.numpy as jnp
from jax import lax
from jax.experimental import pallas as pl
from jax.experimental.pallas import tpu as pltpu
from jax.experimental.pallas import tpu_sc as plsc

N = 50000
E = 800000
NP = 50176
NC, NS = 2, 16
NW = NC * NS
EC = E // NW
B = NP // 128
CAPW = 27904
SENT = int(np.int32(np.uint32((0xFFFF << 16) | N)))
ROWB = 1024
GRID = NP // ROWB


def _sc_mesh():
    return plsc.VectorSubcoreMesh(
        core_axis_name="c", subcore_axis_name="s",
        num_cores=NC, num_subcores=NS)


def _wid():
    return lax.axis_index("s") * NC + lax.axis_index("c")


def _make_bin():
    G = (EC + 15) // 16

    @functools.partial(
        pl.kernel, mesh=_sc_mesh(),
        out_type=(jax.ShapeDtypeStruct((NW, 512), jnp.int32),
                  jax.ShapeDtypeStruct((NW * CAPW,), jnp.int32)),
        scratch_types=[
            pltpu.VMEM((EC + 8,), jnp.int32),
            pltpu.VMEM((EC + 8,), jnp.int32),
            pltpu.VMEM((CAPW,), jnp.int32),
            pltpu.VMEM((512,), jnp.int32),
            pltpu.SMEM((512,), jnp.int32),
            pltpu.SMEM((512,), jnp.int32),
        ],
    )
    def bink(src_hbm, dst_hbm, cnt_hbm, packed_hbm,
             sstage, dstage, wordbuf, cntv, cnt, cur):
        w = _wid()
        lanes = lax.iota(jnp.int32, 16)
        pltpu.sync_copy(src_hbm.at[pl.ds(w * EC, EC)], sstage.at[pl.ds(0, EC)])
        pltpu.sync_copy(dst_hbm.at[pl.ds(w * EC, EC)], dstage.at[pl.ds(0, EC)])

        def zc(j, _):
            cnt[j] = 0
            return 0
        lax.fori_loop(0, 512, zc, 0)

        def count(g, _):
            d16 = dstage[pl.ds(g * 16, 16)]
            valid = (g * 16 + lanes) < EC
            b16 = jnp.where(valid, lax.shift_right_logical(d16, 7), B)
            for lane in range(16):
                bb = b16[lane]
                cnt[bb] = cnt[bb] + 1
            return 0
        lax.fori_loop(0, G, count, 0)

        def mkstart(bb, running):
            cur[bb] = running
            return running + ((cnt[bb] + 7) & ~7)
        lax.fori_loop(0, B + 1, mkstart, jnp.int32(0))

        def emitc(g, _):
            v = jnp.zeros((16,), jnp.int32)
            for lane in range(16):
                v = jnp.where(lanes == lane, cnt[g * 16 + lane], v)
            cntv[pl.ds(g * 16, 16)] = v
            return 0
        lax.fori_loop(0, 32, emitc, 0)
        pltpu.sync_copy(cntv, cnt_hbm.at[w])

        sent = jnp.full((16,), SENT, jnp.int32)

        def fill(j, _):
            wordbuf[pl.ds(j * 16, 16)] = sent
            return 0
        lax.fori_loop(0, CAPW // 16, fill, 0)

        def place(g, _):
            s16 = sstage[pl.ds(g * 16, 16)]
            d16 = dstage[pl.ds(g * 16, 16)]
            valid = (g * 16 + lanes) < EC
            d16 = jnp.where(valid, d16, 0xFFFF)
            s16 = jnp.where(valid, s16, N)
            word = lax.shift_left(d16, 16) | s16
            b16 = jnp.minimum(lax.shift_right_logical(d16, 7), B)
            for lane in range(16):
                bb = b16[lane]
                cu = cur[bb]
                base = cu & ~15
                vec = wordbuf[pl.ds(base, 16)]
                vec = jnp.where(lanes == (cu & 15), word[lane], vec)
                wordbuf[pl.ds(base, 16)] = vec
                cur[bb] = cu + 1
            return 0
        lax.fori_loop(0, G, place, 0)

        pltpu.sync_copy(wordbuf, packed_hbm.at[pl.ds(w * CAPW, CAPW)])

    return bink


def _emit_starts(cbuf, w, st7, ln7):
    def per_tile(t, _):
        def per_group(g, running):
            vec = cbuf[t, pl.ds(g * 16, 16)]
            rv = (vec + 7) & ~7
            for lane in range(16):
                bb = g * 16 + lane

                @pl.when((bb < B) & ((bb & 31) == w))
                def _():
                    kk = bb // 32
                    st7[t * 16 + kk] = running
                    ln7[t * 16 + kk] = vec[lane]
                running = running + rv[lane]
            return running
        lax.fori_loop(0, 32, per_group, jnp.int32(0))
        return 0
    lax.fori_loop(0, NW, per_tile, 0)


def _make_deg():
    CH = 512

    @functools.partial(
        pl.kernel, mesh=_sc_mesh(),
        out_type=jax.ShapeDtypeStruct((NP,), jnp.float32),
        scratch_types=[
            pltpu.VMEM((NW, 512), jnp.int32),
            pltpu.VMEM((CH,), jnp.int32),
            pltpu.VMEM((128,), jnp.float32),
            pltpu.SMEM((512,), jnp.int32),
            pltpu.SMEM((512,), jnp.int32),
            pltpu.SMEM((136,), jnp.int32),
        ],
    )
    def degk(cnt_hbm, packed_hbm, deg_hbm, cbuf, wbuf, degbuf, st7, ln7, dcnt):
        w = _wid()
        lanes = lax.iota(jnp.int32, 16)
        pltpu.sync_copy(cnt_hbm, cbuf)
        _emit_starts(cbuf, w, st7, ln7)

        def per_k(kk, _):
            b = w + 32 * kk

            @pl.when(b < B)
            def _():
                def zz(j, _2):
                    dcnt[j] = 0
                    return 0
                lax.fori_loop(0, 136, zz, 0)

                def per_t(t, _2):
                    st = st7[t * 16 + kk]
                    ln = ln7[t * 16 + kk]
                    nch = (ln + CH - 1) // CH

                    def per_ch(ch, _3):
                        off = pl.multiple_of(t * CAPW + st + ch * CH, 8)
                        pltpu.sync_copy(packed_hbm.at[pl.ds(off, CH)], wbuf)

                        def per_g(g, _4):
                            wv = wbuf[pl.ds(g * 16, 16)]
                            dv = lax.shift_right_logical(wv, 16)
                            ok = lax.shift_right_logical(dv, 7) == b
                            dl = jnp.where(ok, dv & 127, 128)
                            for lane in range(16):
                                dd = dl[lane]
                                dcnt[dd] = dcnt[dd] + 1
                            return 0
                        lax.fori_loop(0, CH // 16, per_g, 0)
                        return 0
                    lax.fori_loop(0, nch, per_ch, 0)
                    return 0
                lax.fori_loop(0, NW, per_t, 0)

                def emit(g, _2):
                    v = jnp.zeros((16,), jnp.int32)
                    for lane in range(16):
                        v = jnp.where(lanes == lane, dcnt[g * 16 + lane], v)
                    degbuf[pl.ds(g * 16, 16)] = v.astype(jnp.float32)
                    return 0
                lax.fori_loop(0, 8, emit, 0)
                pltpu.sync_copy(degbuf, deg_hbm.at[pl.ds(b * 128, 128)])
            return 0
        lax.fori_loop(0, 13, per_k, 0)

    return degk


def _make_prop(F):
    K = 64 if F == 256 else 128
    NB = 2 if F == 256 else 4

    @functools.partial(
        pl.kernel, mesh=_sc_mesh(),
        out_type=jax.ShapeDtypeStruct((NP, F), jnp.float32),
        scratch_types=[
            pltpu.VMEM((NW, 512), jnp.int32),
            pltpu.VMEM((2, K), jnp.int32),
            pltpu.VMEM((2, K), jnp.int32),
            pltpu.VMEM((2, K), jnp.int32),
            pltpu.VMEM((2, K, F), jnp.float32),
            pltpu.VMEM((NB, 136, F), jnp.float32),
            pltpu.SMEM((512,), jnp.int32),
            pltpu.SMEM((512,), jnp.int32),
            pltpu.SemaphoreType.DMA,
            pltpu.SemaphoreType.DMA,
        ],
    )
    def propk(h_hbm, cnt_hbm, packed_hbm, out_hbm,
              cbuf, wbuf, gidx, dlbuf, rowbuf, acc, st7, ln7, sem0, sem1):
        w = _wid()
        lanes = lax.iota(jnp.int32, 16)
        pltpu.sync_copy(cnt_hbm, cbuf)
        _emit_starts(cbuf, w, st7, ln7)
        zf = jnp.zeros((16,), jnp.float32)

        def stage(b, t, kk, ch, p, sem):
            st = st7[t * 16 + kk]
            off = pl.multiple_of(t * CAPW + st + ch * K, 8)
            pltpu.sync_copy(packed_hbm.at[pl.ds(off, K)], wbuf.at[p])

            def unpack(g, _):
                wv = wbuf[p, pl.ds(g * 16, 16)]
                dv = lax.shift_right_logical(wv, 16)
                ok = lax.shift_right_logical(dv, 7) == b
                gidx[p, pl.ds(g * 16, 16)] = jnp.where(
                    ok, wv & 0xFFFF, N + ((g * 16 + lanes) & 127))
                dlbuf[p, pl.ds(g * 16, 16)] = jnp.where(ok, dv & 127, 128)
                return 0
            lax.fori_loop(0, K // 16, unpack, 0)
            pltpu.async_copy(h_hbm.at[gidx.at[p]], rowbuf.at[p], sem)

        def drain_rmw(p, sem):
            pltpu.make_async_copy(
                h_hbm.at[gidx.at[p]], rowbuf.at[p], sem).wait()

            def rmw(g, _):
                dl = dlbuf[p, pl.ds(g * 16, 16)]
                for lane in range(16):
                    r = dl[lane]
                    bank = lane % NB
                    for t2 in range(F // 16):
                        sl = pl.ds(t2 * 16, 16)
                        acc[bank, r, sl] = acc[bank, r, sl] + \
                            rowbuf[p, g * 16 + lane, sl]
                return 0
            lax.fori_loop(0, K // 16, rmw, 0)

        def per_k(kk, _):
            b = w + 32 * kk

            @pl.when(b < B)
            def _():
                def za(j, _2):
                    for bank in range(NB):
                        for t in range(F // 16):
                            acc[bank, j, pl.ds(t * 16, 16)] = zf
                    return 0
                lax.fori_loop(0, 136, za, 0)

                def per_t(t, _2):
                    ln = ln7[t * 16 + kk]
                    nch = (ln + K - 1) // K

                    @pl.when(nch > 0)
                    def _():
                        stage(b, t, kk, 0, 0, sem0)

                        def per_ch(ch, _3):
                            @pl.when(ch % 2 == 0)
                            def _():
                                @pl.when(ch + 1 < nch)
                                def _():
                                    stage(b, t, kk, ch + 1, 1, sem1)
                                drain_rmw(0, sem0)

                            @pl.when(ch % 2 == 1)
                            def _():
                                @pl.when(ch + 1 < nch)
                                def _():
                                    stage(b, t, kk, ch + 1, 0, sem0)
                                drain_rmw(1, sem1)
                            return 0
                        lax.fori_loop(0, nch, per_ch, 0)
                    return 0
                lax.fori_loop(0, NW, per_t, 0)

                def merge(j, _2):
                    for t2 in range(F // 16):
                        sl = pl.ds(t2 * 16, 16)
                        v = acc[0, j, sl]
                        for bank in range(1, NB):
                            v = v + acc[bank, j, sl]
                        acc[0, j, sl] = v
                    return 0
                lax.fori_loop(0, 128, merge, 0)

                pltpu.sync_copy(acc.at[0, pl.ds(0, 128)],
                                out_hbm.at[pl.ds(b * 128, 128)])
            return 0
        lax.fori_loop(0, 13, per_k, 0)

    return propk


def _rowspec(cols):
    return pl.BlockSpec((ROWB, cols), lambda i: (i, 0))


def _full(shape):
    return pl.BlockSpec(shape, lambda i: (0, 0))


def _rowmask(i):
    idx = lax.broadcasted_iota(jnp.int32, (ROWB, 1), 0) + i * ROWB
    return (idx < N).astype(jnp.float32)


def _dis_kernel(deg):
    def body(d_ref, o_ref):
        o_ref[...] = lax.rsqrt(d_ref[...] + 1.0)
    return pl.pallas_call(
        body, grid=(GRID,),
        in_specs=[_rowspec(1)],
        out_specs=_rowspec(1),
        out_shape=jax.ShapeDtypeStruct((NP, 1), jnp.float32))(deg)


def _mm_first(x, W, dis):
    def body(x_ref, w_ref, d_ref, o_ref):
        i = pl.program_id(0)
        h = jnp.dot(x_ref[...], w_ref[...],
                    preferred_element_type=jnp.float32)
        o_ref[...] = h * d_ref[...] * _rowmask(i)
    fi, fo = W.shape
    return pl.pallas_call(
        body, grid=(GRID,),
        in_specs=[_rowspec(fi), _full((fi, fo)), _rowspec(1)],
        out_specs=_rowspec(fo),
        out_shape=jax.ShapeDtypeStruct((NP, fo), jnp.float32))(x, W, dis)


def _mm_mid(accE, h, dis, b, W):
    def body(a_ref, h_ref, d_ref, b_ref, w_ref, o_ref):
        i = pl.program_id(0)
        z = jax.nn.relu((a_ref[...] + h_ref[...]) * d_ref[...] + b_ref[...])
        o_ref[...] = jnp.dot(z, w_ref[...],
                             preferred_element_type=jnp.float32) \
            * d_ref[...] * _rowmask(i)
    fi, fo = W.shape
    return pl.pallas_call(
        body, grid=(GRID,),
        in_specs=[_rowspec(fi), _rowspec(fi), _rowspec(1),
                  _full((1, fi)), _full((fi, fo))],
        out_specs=_rowspec(fo),
        out_shape=jax.ShapeDtypeStruct((NP, fo), jnp.float32))(accE, h, dis, b, W)


def _finish(accE, h, dis, b):
    def body(a_ref, h_ref, d_ref, b_ref, o_ref):
        v = jax.nn.relu(
            (a_ref[...] + h_ref[...]) * d_ref[...] + b_ref[...])
        o_ref[...] = v[:, :64]
    f = h.shape[1]
    return pl.pallas_call(
        body, grid=(GRID,),
        in_specs=[_rowspec(f), _rowspec(f), _rowspec(1), _full((1, f))],
        out_specs=_rowspec(64),
        out_shape=jax.ShapeDtypeStruct((NP, 64), jnp.float32))(accE, h, dis, b)


_bin = _make_bin()
_deg = _make_deg()
_prop256 = _make_prop(256)
_prop128 = _make_prop(128)


def kernel(x, edge_index, W1, b1, W2, b2, W3, b3):
    src = edge_index[0].astype(jnp.int32)
    dst = edge_index[1].astype(jnp.int32)
    xp = jnp.pad(x, ((0, NP - N), (0, 0)))
    W3p = jnp.pad(W3, ((0, 0), (0, 64)))
    b3p = jnp.pad(b3, (0, 64))

    cnts, packed = _bin(src, dst)
    deg = _deg(cnts, packed).reshape(NP, 1)
    dis = _dis_kernel(deg)

    h1 = _mm_first(xp, W1, dis)
    a1 = _prop256(h1, cnts, packed)
    h2 = _mm_mid(a1, h1, dis, b1.reshape(1, -1), W2)
    a2 = _prop128(h2, cnts, packed)
    h3 = _mm_mid(a2, h2, dis, b2.reshape(1, -1), W3p)
    a3 = _prop128(h3, cnts, packed)
    out = _finish(a3, h3, dis, b3p.reshape(1, -1))
    return out[:N]


# speedup vs baseline: 4.5430x; 1.1137x over previous
import functools

import numpy as np
import jax
import jax.numpy as jnp
from jax import lax
from jax.experimental import pallas as pl
from jax.experimental.pallas import tpu as pltpu
from jax.experimental.pallas import tpu_sc as plsc

N = 50000
E = 800000
NP = 50176
NC, NS = 2, 16
NW = NC * NS
EC = E // NW
B = NP // 128
CAPW = 27904
SENT = int(np.int32(np.uint32((0xFFFF << 16) | N)))
ROWB = 1024
GRID = NP // ROWB


def _sc_mesh():
    return plsc.VectorSubcoreMesh(
        core_axis_name="c", subcore_axis_name="s",
        num_cores=NC, num_subcores=NS)


def _wid():
    return lax.axis_index("s") * NC + lax.axis_index("c")


def _make_bin():
    G = (EC + 15) // 16

    @functools.partial(
        pl.kernel, mesh=_sc_mesh(),
        out_type=(jax.ShapeDtypeStruct((NW, 512), jnp.int32),
                  jax.ShapeDtypeStruct((NW * CAPW,), jnp.int32)),
        scratch_types=[
            pltpu.VMEM((EC + 8,), jnp.int32),
            pltpu.VMEM((EC + 8,), jnp.int32),
            pltpu.VMEM((CAPW,), jnp.int32),
            pltpu.VMEM((512,), jnp.int32),
            pltpu.SMEM((512,), jnp.int32),
            pltpu.SMEM((512,), jnp.int32),
        ],
    )
    def bink(src_hbm, dst_hbm, cnt_hbm, packed_hbm,
             sstage, dstage, wordbuf, cntv, cnt, cur):
        w = _wid()
        lanes = lax.iota(jnp.int32, 16)
        pltpu.sync_copy(src_hbm.at[pl.ds(w * EC, EC)], sstage.at[pl.ds(0, EC)])
        pltpu.sync_copy(dst_hbm.at[pl.ds(w * EC, EC)], dstage.at[pl.ds(0, EC)])

        def zc(j, _):
            cnt[j] = 0
            return 0
        lax.fori_loop(0, 512, zc, 0)

        def count(g, _):
            d16 = dstage[pl.ds(g * 16, 16)]
            valid = (g * 16 + lanes) < EC
            b16 = jnp.where(valid, lax.shift_right_logical(d16, 7), B)
            for lane in range(16):
                bb = b16[lane]
                cnt[bb] = cnt[bb] + 1
            return 0
        lax.fori_loop(0, G, count, 0)

        def mkstart(bb, running):
            cur[bb] = running
            return running + ((cnt[bb] + 7) & ~7)
        lax.fori_loop(0, B + 1, mkstart, jnp.int32(0))

        def emitc(g, _):
            v = jnp.zeros((16,), jnp.int32)
            for lane in range(16):
                v = jnp.where(lanes == lane, cnt[g * 16 + lane], v)
            cntv[pl.ds(g * 16, 16)] = v
            return 0
        lax.fori_loop(0, 32, emitc, 0)
        pltpu.sync_copy(cntv, cnt_hbm.at[w])

        sent = jnp.full((16,), SENT, jnp.int32)

        def fill(j, _):
            wordbuf[pl.ds(j * 16, 16)] = sent
            return 0
        lax.fori_loop(0, CAPW // 16, fill, 0)

        def place(g, _):
            s16 = sstage[pl.ds(g * 16, 16)]
            d16 = dstage[pl.ds(g * 16, 16)]
            valid = (g * 16 + lanes) < EC
            d16 = jnp.where(valid, d16, 0xFFFF)
            s16 = jnp.where(valid, s16, N)
            word = lax.shift_left(d16, 16) | s16
            b16 = jnp.minimum(lax.shift_right_logical(d16, 7), B)
            for lane in range(16):
                bb = b16[lane]
                cu = cur[bb]
                base = cu & ~15
                vec = wordbuf[pl.ds(base, 16)]
                vec = jnp.where(lanes == (cu & 15), word[lane], vec)
                wordbuf[pl.ds(base, 16)] = vec
                cur[bb] = cu + 1
            return 0
        lax.fori_loop(0, G, place, 0)

        pltpu.sync_copy(wordbuf, packed_hbm.at[pl.ds(w * CAPW, CAPW)])

    return bink


def _emit_starts(cbuf, w, st7, ln7):
    def per_tile(t, _):
        def per_group(g, running):
            vec = cbuf[t, pl.ds(g * 16, 16)]
            rv = (vec + 7) & ~7
            for lane in range(16):
                bb = g * 16 + lane

                @pl.when((bb < B) & ((bb & 31) == w))
                def _():
                    kk = bb // 32
                    st7[t * 16 + kk] = running
                    ln7[t * 16 + kk] = vec[lane]
                running = running + rv[lane]
            return running
        lax.fori_loop(0, 32, per_group, jnp.int32(0))
        return 0
    lax.fori_loop(0, NW, per_tile, 0)


def _make_deg():
    CH = 512

    @functools.partial(
        pl.kernel, mesh=_sc_mesh(),
        out_type=jax.ShapeDtypeStruct((NP,), jnp.float32),
        scratch_types=[
            pltpu.VMEM((NW, 512), jnp.int32),
            pltpu.VMEM((CH,), jnp.int32),
            pltpu.VMEM((128,), jnp.float32),
            pltpu.SMEM((512,), jnp.int32),
            pltpu.SMEM((512,), jnp.int32),
            pltpu.SMEM((136,), jnp.int32),
        ],
    )
    def degk(cnt_hbm, packed_hbm, deg_hbm, cbuf, wbuf, degbuf, st7, ln7, dcnt):
        w = _wid()
        lanes = lax.iota(jnp.int32, 16)
        pltpu.sync_copy(cnt_hbm, cbuf)
        _emit_starts(cbuf, w, st7, ln7)

        def per_k(kk, _):
            b = w + 32 * kk

            @pl.when(b < B)
            def _():
                def zz(j, _2):
                    dcnt[j] = 0
                    return 0
                lax.fori_loop(0, 136, zz, 0)

                def per_t(t, _2):
                    st = st7[t * 16 + kk]
                    ln = ln7[t * 16 + kk]
                    nch = (ln + CH - 1) // CH

                    def per_ch(ch, _3):
                        off = pl.multiple_of(t * CAPW + st + ch * CH, 8)
                        pltpu.sync_copy(packed_hbm.at[pl.ds(off, CH)], wbuf)

                        def per_g(g, _4):
                            wv = wbuf[pl.ds(g * 16, 16)]
                            dv = lax.shift_right_logical(wv, 16)
                            ok = lax.shift_right_logical(dv, 7) == b
                            dl = jnp.where(ok, dv & 127, 128)
                            for lane in range(16):
                                dd = dl[lane]
                                dcnt[dd] = dcnt[dd] + 1
                            return 0
                        lax.fori_loop(0, CH // 16, per_g, 0)
                        return 0
                    lax.fori_loop(0, nch, per_ch, 0)
                    return 0
                lax.fori_loop(0, NW, per_t, 0)

                def emit(g, _2):
                    v = jnp.zeros((16,), jnp.int32)
                    for lane in range(16):
                        v = jnp.where(lanes == lane, dcnt[g * 16 + lane], v)
                    degbuf[pl.ds(g * 16, 16)] = v.astype(jnp.float32)
                    return 0
                lax.fori_loop(0, 8, emit, 0)
                pltpu.sync_copy(degbuf, deg_hbm.at[pl.ds(b * 128, 128)])
            return 0
        lax.fori_loop(0, 13, per_k, 0)

    return degk


def _make_prop(F):
    K = 64 if F == 256 else 128
    NB = 2 if F == 256 else 4

    @functools.partial(
        pl.kernel, mesh=_sc_mesh(),
        out_type=jax.ShapeDtypeStruct((NP, F), jnp.float32),
        scratch_types=[
            pltpu.VMEM((NW, 512), jnp.int32),
            pltpu.VMEM((2, K), jnp.int32),
            pltpu.VMEM((2, K), jnp.int32),
            pltpu.VMEM((2, K), jnp.int32),
            pltpu.VMEM((2, K, F), jnp.float32),
            pltpu.VMEM((NB, 136, F), jnp.float32),
            pltpu.SMEM((512,), jnp.int32),
            pltpu.SMEM((512,), jnp.int32),
            pltpu.SemaphoreType.DMA,
            pltpu.SemaphoreType.DMA,
        ],
    )
    def propk(h_hbm, cnt_hbm, packed_hbm, out_hbm,
              cbuf, wbuf, gidx, dlbuf, rowbuf, acc, st7, ln7, sem0, sem1):
        w = _wid()
        lanes = lax.iota(jnp.int32, 16)
        pltpu.sync_copy(cnt_hbm, cbuf)
        _emit_starts(cbuf, w, st7, ln7)
        zf = jnp.zeros((16,), jnp.float32)

        def stage(b, t, kk, ch, p, sem):
            st = st7[t * 16 + kk]
            off = pl.multiple_of(t * CAPW + st + ch * K, 8)
            pltpu.sync_copy(packed_hbm.at[pl.ds(off, K)], wbuf.at[p])

            def unpack(g, _):
                wv = wbuf[p, pl.ds(g * 16, 16)]
                dv = lax.shift_right_logical(wv, 16)
                ok = lax.shift_right_logical(dv, 7) == b
                gidx[p, pl.ds(g * 16, 16)] = jnp.where(
                    ok, wv & 0xFFFF, N + ((g * 16 + lanes) & 127))
                dlbuf[p, pl.ds(g * 16, 16)] = jnp.where(ok, dv & 127, 128)
                return 0
            lax.fori_loop(0, K // 16, unpack, 0)
            pltpu.async_copy(h_hbm.at[gidx.at[p]], rowbuf.at[p], sem)

        def drain_rmw(p, sem):
            pltpu.make_async_copy(
                h_hbm.at[gidx.at[p]], rowbuf.at[p], sem).wait()

            def rmw(g, _):
                dl = dlbuf[p, pl.ds(g * 16, 16)]
                for lane in range(16):
                    r = dl[lane]
                    bank = lane % NB
                    for t2 in range(F // 16):
                        sl = pl.ds(t2 * 16, 16)
                        acc[bank, r, sl] = acc[bank, r, sl] + \
                            rowbuf[p, g * 16 + lane, sl]
                return 0
            lax.fori_loop(0, K // 16, rmw, 0)

        def per_k(kk, _):
            b = w + 32 * kk

            @pl.when(b < B)
            def _():
                def za(j, _2):
                    for bank in range(NB):
                        for t in range(F // 16):
                            acc[bank, j, pl.ds(t * 16, 16)] = zf
                    return 0
                lax.fori_loop(0, 136, za, 0)

                stage(b, 0, kk, 0, 0, sem0)

                def per_t(t, _2):
                    ln = ln7[t * 16 + kk]
                    nch = (ln + K - 1) // K

                    def headstage(p):
                        @pl.when(t + 1 < NW)
                        def _():
                            stage(b, t + 1, kk, 0, p, [sem0, sem1][p])

                    def extras(p, sem):
                        def per_ch(ch, _3):
                            stage(b, t, kk, ch, p, sem)
                            drain_rmw(p, sem)
                            return 0
                        lax.fori_loop(1, nch, per_ch, 0)

                    @pl.when(t % 2 == 0)
                    def _():
                        headstage(1)
                        drain_rmw(0, sem0)
                        extras(0, sem0)

                    @pl.when(t % 2 == 1)
                    def _():
                        headstage(0)
                        drain_rmw(1, sem1)
                        extras(1, sem1)
                    return 0
                lax.fori_loop(0, NW, per_t, 0)

                def merge(j, _2):
                    for t2 in range(F // 16):
                        sl = pl.ds(t2 * 16, 16)
                        v = acc[0, j, sl]
                        for bank in range(1, NB):
                            v = v + acc[bank, j, sl]
                        acc[0, j, sl] = v
                    return 0
                lax.fori_loop(0, 128, merge, 0)

                pltpu.sync_copy(acc.at[0, pl.ds(0, 128)],
                                out_hbm.at[pl.ds(b * 128, 128)])
            return 0
        lax.fori_loop(0, 13, per_k, 0)

    return propk


def _rowspec(cols):
    return pl.BlockSpec((ROWB, cols), lambda i: (i, 0))


def _full(shape):
    return pl.BlockSpec(shape, lambda i: (0, 0))


def _rowmask(i):
    idx = lax.broadcasted_iota(jnp.int32, (ROWB, 1), 0) + i * ROWB
    return (idx < N).astype(jnp.float32)


def _dis_kernel(deg):
    def body(d_ref, o_ref):
        o_ref[...] = lax.rsqrt(d_ref[...] + 1.0)
    return pl.pallas_call(
        body, grid=(GRID,),
        in_specs=[_rowspec(1)],
        out_specs=_rowspec(1),
        out_shape=jax.ShapeDtypeStruct((NP, 1), jnp.float32))(deg)


def _mm_first(x, W, dis):
    def body(x_ref, w_ref, d_ref, o_ref):
        i = pl.program_id(0)
        h = jnp.dot(x_ref[...], w_ref[...],
                    preferred_element_type=jnp.float32)
        o_ref[...] = h * d_ref[...] * _rowmask(i)
    fi, fo = W.shape
    return pl.pallas_call(
        body, grid=(GRID,),
        in_specs=[_rowspec(fi), _full((fi, fo)), _rowspec(1)],
        out_specs=_rowspec(fo),
        out_shape=jax.ShapeDtypeStruct((NP, fo), jnp.float32))(x, W, dis)


def _mm_mid(accE, h, dis, b, W):
    def body(a_ref, h_ref, d_ref, b_ref, w_ref, o_ref):
        i = pl.program_id(0)
        z = jax.nn.relu((a_ref[...] + h_ref[...]) * d_ref[...] + b_ref[...])
        o_ref[...] = jnp.dot(z, w_ref[...],
                             preferred_element_type=jnp.float32) \
            * d_ref[...] * _rowmask(i)
    fi, fo = W.shape
    return pl.pallas_call(
        body, grid=(GRID,),
        in_specs=[_rowspec(fi), _rowspec(fi), _rowspec(1),
                  _full((1, fi)), _full((fi, fo))],
        out_specs=_rowspec(fo),
        out_shape=jax.ShapeDtypeStruct((NP, fo), jnp.float32))(accE, h, dis, b, W)


def _finish(accE, h, dis, b):
    def body(a_ref, h_ref, d_ref, b_ref, o_ref):
        v = jax.nn.relu(
            (a_ref[...] + h_ref[...]) * d_ref[...] + b_ref[...])
        o_ref[...] = v[:, :64]
    f = h.shape[1]
    return pl.pallas_call(
        body, grid=(GRID,),
        in_specs=[_rowspec(f), _rowspec(f), _rowspec(1), _full((1, f))],
        out_specs=_rowspec(64),
        out_shape=jax.ShapeDtypeStruct((NP, 64), jnp.float32))(accE, h, dis, b)


_bin = _make_bin()
_deg = _make_deg()
_prop256 = _make_prop(256)
_prop128 = _make_prop(128)


def kernel(x, edge_index, W1, b1, W2, b2, W3, b3):
    src = edge_index[0].astype(jnp.int32)
    dst = edge_index[1].astype(jnp.int32)
    xp = jnp.pad(x, ((0, NP - N), (0, 0)))
    W3p = jnp.pad(W3, ((0, 0), (0, 64)))
    b3p = jnp.pad(b3, (0, 64))

    cnts, packed = _bin(src, dst)
    deg = _deg(cnts, packed).reshape(NP, 1)
    dis = _dis_kernel(deg)

    h1 = _mm_first(xp, W1, dis)
    a1 = _prop256(h1, cnts, packed)
    h2 = _mm_mid(a1, h1, dis, b1.reshape(1, -1), W2)
    a2 = _prop128(h2, cnts, packed)
    h3 = _mm_mid(a2, h2, dis, b2.reshape(1, -1), W3p)
    a3 = _prop128(h3, cnts, packed)
    out = _finish(a3, h3, dis, b3p.reshape(1, -1))
    return out[:N]
